# baseline (speedup 1.0000x reference)
import functools

import jax
import jax.numpy as jnp
from jax import lax
from jax.experimental import pallas as pl
from jax.experimental.pallas import tpu as pltpu
from jax.experimental.pallas import tpu_sc as plsc

NUM_CORES = 2
NUM_SUBCORES = 16
NUM_WORKERS = NUM_CORES * NUM_SUBCORES

CHUNK = 640
STREAM = 128


@functools.partial(jax.jit, static_argnums=(2, 3))
def _sc_gather(flat_ids, table, b_total, d):
    b_per_w = b_total // NUM_WORKERS
    n_chunks = b_per_w // CHUNK
    n_pairs = n_chunks // 2
    n_streams = CHUNK // STREAM
    mesh = plsc.VectorSubcoreMesh(core_axis_name="c", subcore_axis_name="s")

    @functools.partial(
        pl.kernel,
        mesh=mesh,
        out_type=jax.ShapeDtypeStruct((b_total, d), jnp.float32),
        scratch_types=[
            pltpu.VMEM((b_per_w // STREAM, STREAM), jnp.int32),
            pltpu.VMEM((CHUNK, d), jnp.float32),
            pltpu.VMEM((CHUNK, d), jnp.float32),
            pltpu.SemaphoreType.DMA,
            pltpu.SemaphoreType.DMA,
            pltpu.SemaphoreType.DMA,
            pltpu.SemaphoreType.DMA,
        ],
        compiler_params=pltpu.CompilerParams(use_tc_tiling_on_sc=False),
    )
    def k(ids_hbm, table_hbm, out_hbm, ids_v, rows0, rows1, g0, g1, o0, o1):
        wid = lax.axis_index("s") * NUM_CORES + lax.axis_index("c")
        base = wid * b_per_w
        rows_per_w = b_per_w // STREAM
        pltpu.sync_copy(ids_hbm.at[pl.ds(wid * rows_per_w, rows_per_w)], ids_v)

        def fire(slot, ch, gsem):
            for j in range(n_streams):
                pltpu.async_copy(
                    table_hbm.at[ids_v.at[ch * n_streams + j]],
                    slot.at[pl.ds(j * STREAM, STREAM)],
                    gsem,
                )

        def drain(slot, ch, gsem):
            for j in range(n_streams):
                pltpu.make_async_copy(
                    table_hbm.at[ids_v.at[ch * n_streams + j]],
                    slot.at[pl.ds(j * STREAM, STREAM)],
                    gsem,
                ).wait()

        def store(slot, ch, osem):
            pltpu.async_copy(
                slot, out_hbm.at[pl.ds(base + ch * CHUNK, CHUNK)], osem
            )

        def wait_store(slot, ch, osem):
            pltpu.make_async_copy(
                slot, out_hbm.at[pl.ds(base + ch * CHUNK, CHUNK)], osem
            ).wait()

        def body(i, carry):
            c0 = 2 * i
            c1 = 2 * i + 1

            @pl.when(i > 0)
            def _():
                wait_store(rows0, c0 - 2, o0)

            fire(rows0, c0, g0)

            @pl.when(i > 0)
            def _():
                wait_store(rows1, c1 - 2, o1)

            fire(rows1, c1, g1)
            drain(rows0, c0, g0)
            store(rows0, c0, o0)
            drain(rows1, c1, g1)
            store(rows1, c1, o1)
            return carry

        lax.fori_loop(0, n_pairs, body, 0)
        wait_store(rows0, n_chunks - 2, o0)
        wait_store(rows1, n_chunks - 1, o1)

    return k(flat_ids, table)


SLAB = 128


@jax.jit
def _sc_transpose(table_t, tail_t):
    d, v = table_t.shape
    n_slabs = v // SLAB
    per_w = (n_slabs + NUM_WORKERS - 1) // NUM_WORKERS
    mesh = plsc.VectorSubcoreMesh(core_axis_name="c", subcore_axis_name="s")

    @functools.partial(
        pl.kernel,
        mesh=mesh,
        out_type=jax.ShapeDtypeStruct((v // 2, 2 * d), jnp.float32),
        scratch_types=[
            pltpu.VMEM((d, SLAB + 1), jnp.float32),
            pltpu.VMEM((d, SLAB + 1), jnp.float32),
            pltpu.VMEM((SLAB // 2, 2 * d), jnp.float32),
            pltpu.VMEM((SLAB // 2, 2 * d), jnp.float32),
            pltpu.VMEM((d, SLAB // 2), jnp.float32),
            pltpu.SemaphoreType.DMA,
            pltpu.SemaphoreType.DMA,
            pltpu.SemaphoreType.DMA,
            pltpu.SemaphoreType.DMA,
        ],
        compiler_params=pltpu.CompilerParams(
            use_tc_tiling_on_sc=True, needs_layout_passes=False),
    )
    def tk(tt_hbm, tail_hbm, out_hbm, in0, in1, ot0, ot1, tbuf,
           gi0, gi1, go0, go1):
        wid = lax.axis_index("s") * NUM_CORES + lax.axis_index("c")
        iota16 = lax.iota(jnp.int32, 16)

        def slab_of(t):
            return wid + NUM_WORKERS * t

        def fire_in(buf, c, sem):
            pltpu.async_copy(
                tt_hbm.at[:, pl.ds(c * SLAB, SLAB)],
                buf.at[:, pl.ds(0, SLAB)], sem)

        def wait_in(buf, c, sem):
            pltpu.make_async_copy(
                tt_hbm.at[:, pl.ds(c * SLAB, SLAB)],
                buf.at[:, pl.ds(0, SLAB)], sem).wait()

        def transpose(buf, obuf, n_kp):
            def row(kp, carry):
                col = jnp.full((16,), kp, jnp.int32)
                for q in range(d // 16):
                    vv = plsc.load_gather(buf, [iota16 + q * 16, col])
                    obuf[kp // 2, pl.ds((kp % 2) * d + q * 16, 16)] = vv
                return carry
            lax.fori_loop(0, n_kp, row, 0)

        def fire_out(obuf, c, sem):
            pltpu.async_copy(
                obuf, out_hbm.at[pl.ds(c * (SLAB // 2), SLAB // 2)], sem)

        def wait_out(obuf, c, sem):
            pltpu.make_async_copy(
                obuf, out_hbm.at[pl.ds(c * (SLAB // 2), SLAB // 2)], sem).wait()

        def body(i, carry):
            c0 = slab_of(2 * i)
            c1 = slab_of(2 * i + 1)

            @pl.when(c0 < n_slabs)
            def _():
                @pl.when(i > 0)
                def _():
                    wait_out(ot0, slab_of(2 * i - 2), go0)
                fire_in(in0, c0, gi0)

            @pl.when(c1 < n_slabs)
            def _():
                @pl.when(i > 0)
                def _():
                    wait_out(ot1, slab_of(2 * i - 1), go1)
                fire_in(in1, c1, gi1)

            @pl.when(c0 < n_slabs)
            def _():
                wait_in(in0, c0, gi0)
                transpose(in0, ot0, SLAB)
                fire_out(ot0, c0, go0)

            @pl.when(c1 < n_slabs)
            def _():
                wait_in(in1, c1, gi1)
                transpose(in1, ot1, SLAB)
                fire_out(ot1, c1, go1)
            return carry

        n_iter = (per_w + 1) // 2
        lax.fori_loop(0, n_iter, body, 0)

        last0 = slab_of(2 * n_iter - 2)
        last1 = slab_of(2 * n_iter - 1)

        @pl.when(last0 < n_slabs)
        def _():
            wait_out(ot0, last0, go0)

        @pl.when(last1 < n_slabs)
        def _():
            wait_out(ot1, last1, go1)

        @pl.when(wid == 0)
        def _():
            pltpu.sync_copy(tail_hbm, tbuf)
            transpose(tbuf, ot0, SLAB // 2)
            pltpu.sync_copy(
                ot0.at[pl.ds(0, SLAB // 4)],
                out_hbm.at[pl.ds(n_slabs * (SLAB // 2), SLAB // 4)])

    return tk(table_t, tail_t)


def kernel(token_ids, embeddings):
    b_total = token_ids.shape[0] * token_ids.shape[1]
    d = embeddings.shape[1]
    flat = token_ids.reshape(b_total // STREAM, STREAM).astype(jnp.int32)
    n_full = (embeddings.shape[0] // SLAB) * SLAB
    table_lin = _sc_transpose(embeddings.T, embeddings[n_full:].T)
    table2 = table_lin.reshape(embeddings.shape)
    out = _sc_gather(flat, table2, b_total, d)
    return out.reshape(token_ids.shape + (d,))


# speedup vs baseline: 1.1349x; 1.1349x over previous
import functools

import jax
import jax.numpy as jnp
from jax import lax
from jax.experimental import pallas as pl
from jax.experimental.pallas import tpu as pltpu
from jax.experimental.pallas import tpu_sc as plsc

NUM_CORES = 2
NUM_SUBCORES = 16
NUM_WORKERS = NUM_CORES * NUM_SUBCORES

CHUNK = 640
STREAM = 128


@functools.partial(jax.jit, static_argnums=(2, 3))
def _sc_gather(flat_ids, table, b_total, d):
    b_per_w = b_total // NUM_WORKERS
    n_chunks = b_per_w // CHUNK
    n_pairs = n_chunks // 2
    n_streams = CHUNK // STREAM
    mesh = plsc.VectorSubcoreMesh(core_axis_name="c", subcore_axis_name="s")

    @functools.partial(
        pl.kernel,
        mesh=mesh,
        out_type=jax.ShapeDtypeStruct((b_total, d), jnp.float32),
        scratch_types=[
            pltpu.VMEM((b_per_w // STREAM, STREAM), jnp.int32),
            pltpu.VMEM((CHUNK, d), jnp.float32),
            pltpu.VMEM((CHUNK, d), jnp.float32),
            pltpu.SemaphoreType.DMA,
            pltpu.SemaphoreType.DMA,
            pltpu.SemaphoreType.DMA,
            pltpu.SemaphoreType.DMA,
        ],
        compiler_params=pltpu.CompilerParams(use_tc_tiling_on_sc=False),
    )
    def k(ids_hbm, table_hbm, out_hbm, ids_v, rows0, rows1, g0, g1, o0, o1):
        wid = lax.axis_index("s") * NUM_CORES + lax.axis_index("c")
        base = wid * b_per_w
        rows_per_w = b_per_w // STREAM
        pltpu.sync_copy(ids_hbm.at[pl.ds(wid * rows_per_w, rows_per_w)], ids_v)

        def fire(slot, ch, gsem):
            for j in range(n_streams):
                pltpu.async_copy(
                    table_hbm.at[ids_v.at[ch * n_streams + j]],
                    slot.at[pl.ds(j * STREAM, STREAM)],
                    gsem,
                )

        def drain(slot, ch, gsem):
            for j in range(n_streams):
                pltpu.make_async_copy(
                    table_hbm.at[ids_v.at[ch * n_streams + j]],
                    slot.at[pl.ds(j * STREAM, STREAM)],
                    gsem,
                ).wait()

        def store(slot, ch, osem):
            pltpu.async_copy(
                slot, out_hbm.at[pl.ds(base + ch * CHUNK, CHUNK)], osem
            )

        def wait_store(slot, ch, osem):
            pltpu.make_async_copy(
                slot, out_hbm.at[pl.ds(base + ch * CHUNK, CHUNK)], osem
            ).wait()

        def body(i, carry):
            c0 = 2 * i
            c1 = 2 * i + 1

            @pl.when(i > 0)
            def _():
                wait_store(rows0, c0 - 2, o0)

            fire(rows0, c0, g0)

            @pl.when(i > 0)
            def _():
                wait_store(rows1, c1 - 2, o1)

            fire(rows1, c1, g1)
            drain(rows0, c0, g0)
            store(rows0, c0, o0)
            drain(rows1, c1, g1)
            store(rows1, c1, o1)
            return carry

        lax.fori_loop(0, n_pairs, body, 0)
        wait_store(rows0, n_chunks - 2, o0)
        wait_store(rows1, n_chunks - 1, o1)

    return k(flat_ids, table)


SLAB = 128


@jax.jit
def _sc_transpose(table_t, tail_t):
    d, v = table_t.shape
    n_slabs = v // SLAB
    per_w = (n_slabs + NUM_WORKERS - 1) // NUM_WORKERS
    mesh = plsc.VectorSubcoreMesh(core_axis_name="c", subcore_axis_name="s")

    @functools.partial(
        pl.kernel,
        mesh=mesh,
        out_type=jax.ShapeDtypeStruct((v // 2, 2 * d), jnp.float32),
        scratch_types=[
            pltpu.VMEM((d, SLAB + 1), jnp.float32),
            pltpu.VMEM((d, SLAB + 1), jnp.float32),
            pltpu.VMEM((SLAB // 2, 2 * d + 1), jnp.float32),
            pltpu.VMEM((SLAB // 2, 2 * d + 1), jnp.float32),
            pltpu.VMEM((d, SLAB // 2), jnp.float32),
            pltpu.SemaphoreType.DMA,
            pltpu.SemaphoreType.DMA,
            pltpu.SemaphoreType.DMA,
            pltpu.SemaphoreType.DMA,
        ],
        compiler_params=pltpu.CompilerParams(
            use_tc_tiling_on_sc=True, needs_layout_passes=False),
    )
    def tk(tt_hbm, tail_hbm, out_hbm, in0, in1, ot0, ot1, tbuf,
           gi0, gi1, go0, go1):
        wid = lax.axis_index("s") * NUM_CORES + lax.axis_index("c")
        iota16 = lax.iota(jnp.int32, 16)

        def slab_of(t):
            return wid + NUM_WORKERS * t

        def fire_in(buf, c, sem):
            pltpu.async_copy(
                tt_hbm.at[:, pl.ds(c * SLAB, SLAB)],
                buf.at[:, pl.ds(0, SLAB)], sem)

        def wait_in(buf, c, sem):
            pltpu.make_async_copy(
                tt_hbm.at[:, pl.ds(c * SLAB, SLAB)],
                buf.at[:, pl.ds(0, SLAB)], sem).wait()

        alt64 = (iota16 & 1) * d
        rowv = [lax.shift_right_logical(iota16 + 16 * q, 1)
                for q in range(SLAB // 16)]

        def transpose(buf, obuf, n_kp):
            def col(j, carry):
                cvec = alt64 + j
                for q in range(n_kp // 16):
                    vv = buf[j, pl.ds(q * 16, 16)]
                    plsc.store_scatter(obuf, [rowv[q], cvec], vv)
                return carry
            lax.fori_loop(0, d, col, 0)

        def fire_out(obuf, c, sem):
            pltpu.async_copy(
                obuf.at[:, pl.ds(0, 2 * d)],
                out_hbm.at[pl.ds(c * (SLAB // 2), SLAB // 2)], sem)

        def wait_out(obuf, c, sem):
            pltpu.make_async_copy(
                obuf.at[:, pl.ds(0, 2 * d)],
                out_hbm.at[pl.ds(c * (SLAB // 2), SLAB // 2)], sem).wait()

        def body(i, carry):
            c0 = slab_of(2 * i)
            c1 = slab_of(2 * i + 1)

            @pl.when(c0 < n_slabs)
            def _():
                @pl.when(i > 0)
                def _():
                    wait_out(ot0, slab_of(2 * i - 2), go0)
                fire_in(in0, c0, gi0)

            @pl.when(c1 < n_slabs)
            def _():
                @pl.when(i > 0)
                def _():
                    wait_out(ot1, slab_of(2 * i - 1), go1)
                fire_in(in1, c1, gi1)

            @pl.when(c0 < n_slabs)
            def _():
                wait_in(in0, c0, gi0)
                transpose(in0, ot0, SLAB)
                fire_out(ot0, c0, go0)

            @pl.when(c1 < n_slabs)
            def _():
                wait_in(in1, c1, gi1)
                transpose(in1, ot1, SLAB)
                fire_out(ot1, c1, go1)
            return carry

        n_iter = (per_w + 1) // 2
        lax.fori_loop(0, n_iter, body, 0)

        last0 = slab_of(2 * n_iter - 2)
        last1 = slab_of(2 * n_iter - 1)

        @pl.when(last0 < n_slabs)
        def _():
            wait_out(ot0, last0, go0)

        @pl.when(last1 < n_slabs)
        def _():
            wait_out(ot1, last1, go1)

        @pl.when(wid == 0)
        def _():
            pltpu.sync_copy(tail_hbm, tbuf)
            transpose(tbuf, ot0, SLAB // 2)
            pltpu.sync_copy(
                ot0.at[pl.ds(0, SLAB // 4), pl.ds(0, 2 * d)],
                out_hbm.at[pl.ds(n_slabs * (SLAB // 2), SLAB // 4)])

    return tk(table_t, tail_t)


def kernel(token_ids, embeddings):
    b_total = token_ids.shape[0] * token_ids.shape[1]
    d = embeddings.shape[1]
    flat = token_ids.reshape(b_total // STREAM, STREAM).astype(jnp.int32)
    n_full = (embeddings.shape[0] // SLAB) * SLAB
    table_lin = _sc_transpose(embeddings.T, embeddings[n_full:].T)
    table2 = table_lin.reshape(embeddings.shape)
    out = _sc_gather(flat, table2, b_total, d)
    return out.reshape(token_ids.shape + (d,))
